# baseline (speedup 1.0000x reference)
import functools

import jax
import jax.numpy as jnp
from jax.experimental import pallas as pl

_TOKENS = None


def _classifier_kernel(x_ref, w1_ref, b1_ref, g1_ref, be1_ref,
                       w2_ref, b2_ref, g2_ref, be2_ref,
                       w3_ref, b3_ref, idx_ref):
    x = x_ref[...]
    h = jnp.dot(x, w1_ref[...], preferred_element_type=jnp.float32)
    h = h + b1_ref[...]
    mu = jnp.mean(h, axis=0, keepdims=True)
    var = jnp.mean((h - mu) ** 2, axis=0, keepdims=True)
    h = (h - mu) / jnp.sqrt(var + 1e-5) * g1_ref[...] + be1_ref[...]
    h = jnp.maximum(h, 0.0)
    h = jnp.dot(h, w2_ref[...], preferred_element_type=jnp.float32)
    h = h + b2_ref[...]
    mu = jnp.mean(h, axis=0, keepdims=True)
    var = jnp.mean((h - mu) ** 2, axis=0, keepdims=True)
    h = (h - mu) / jnp.sqrt(var + 1e-5) * g2_ref[...] + be2_ref[...]
    h = jnp.maximum(h, 0.0)
    v = jnp.sum(h * w3_ref[...], axis=1, keepdims=True) + b3_ref[...]
    z = jax.nn.sigmoid(v)
    idx = jnp.clip(jnp.round(z), 0.0, 1.0)
    idx_ref[...] = idx.astype(jnp.int32)


def _heads_kernel(x_ref, idx_ref, wp_ref, bp_ref, out_ref):
    xb = x_ref[...]
    o0 = jnp.dot(xb, wp_ref[0], preferred_element_type=jnp.float32)
    o1 = jnp.dot(xb, wp_ref[1], preferred_element_type=jnp.float32)
    m = (idx_ref[...] > 0)
    bp0 = bp_ref[0:1, :]
    bp1 = bp_ref[1:2, :]
    out_ref[...] = jnp.where(m, o1 + bp1, o0 + bp0)


def kernel(x, W1, b1, g1, be1, W2, b2, g2, be2, W3, b3, Wp, bp):
    Bx, Nx, D = x.shape
    T = Bx * Nx
    C, _, P = Wp.shape
    xr = x.reshape(T, D)

    idx = pl.pallas_call(
        _classifier_kernel,
        out_shape=jax.ShapeDtypeStruct((T, 1), jnp.int32),
    )(xr, W1, b1.reshape(1, -1), g1.reshape(1, -1), be1.reshape(1, -1),
      W2, b2.reshape(1, -1), g2.reshape(1, -1), be2.reshape(1, -1),
      W3.reshape(1, -1), b3.reshape(1, -1))

    xb16 = xr.astype(jnp.bfloat16)
    wp16 = Wp.astype(jnp.bfloat16)

    TB = 256
    out = pl.pallas_call(
        _heads_kernel,
        grid=(T // TB,),
        in_specs=[
            pl.BlockSpec((TB, D), lambda i: (i, 0)),
            pl.BlockSpec((TB, 1), lambda i: (i, 0)),
            pl.BlockSpec((C, D, P), lambda i: (0, 0, 0)),
            pl.BlockSpec((C, P), lambda i: (0, 0)),
        ],
        out_specs=pl.BlockSpec((TB, P), lambda i: (i, 0)),
        out_shape=jax.ShapeDtypeStruct((T, P), jnp.float32),
    )(xb16, idx, wp16, bp)

    return out.reshape(Bx, Nx, P)


# speedup vs baseline: 1.1071x; 1.1071x over previous
import jax
import jax.numpy as jnp
from jax.experimental import pallas as pl
from jax.experimental.pallas import tpu as pltpu


def _classifier_kernel(x_ref, w1_ref, b1_ref, g1_ref, be1_ref,
                       w2_ref, b2_ref, g2_ref, be2_ref,
                       w3_ref, b3_ref, xb_ref, idx_ref, h1_ref):
    i = pl.program_id(0)
    nsteps = pl.num_programs(0)
    tb = x_ref.shape[0]
    x16 = x_ref[...].astype(jnp.bfloat16)
    xb_ref[...] = x16
    h1 = jnp.dot(x16, w1_ref[...].astype(jnp.bfloat16),
                 preferred_element_type=jnp.float32)
    h1_ref[pl.ds(i * tb, tb), :] = h1

    @pl.when(i == nsteps - 1)
    def _finish():
        h = h1_ref[...] + b1_ref[...]
        mu = jnp.mean(h, axis=0, keepdims=True)
        var = jnp.mean((h - mu) ** 2, axis=0, keepdims=True)
        h = (h - mu) / jnp.sqrt(var + 1e-5) * g1_ref[...] + be1_ref[...]
        h = jnp.maximum(h, 0.0)
        h = jnp.dot(h.astype(jnp.bfloat16), w2_ref[...].astype(jnp.bfloat16),
                    preferred_element_type=jnp.float32)
        h = h + b2_ref[...]
        mu = jnp.mean(h, axis=0, keepdims=True)
        var = jnp.mean((h - mu) ** 2, axis=0, keepdims=True)
        h = (h - mu) / jnp.sqrt(var + 1e-5) * g2_ref[...] + be2_ref[...]
        h = jnp.maximum(h, 0.0)
        h16 = h.astype(jnp.bfloat16).astype(jnp.float32)
        w3 = w3_ref[...].astype(jnp.bfloat16).astype(jnp.float32)
        v = jnp.sum(h16 * w3, axis=1, keepdims=True) + b3_ref[...]
        z = jax.nn.sigmoid(v)
        idx_ref[...] = jnp.clip(jnp.round(z), 0.0, 1.0).astype(jnp.int32)


def _heads_kernel(xb_ref, idx_ref, wp_ref, bp_ref, out_ref, wp16_ref):
    i = pl.program_id(0)

    @pl.when(i == 0)
    def _cast():
        wp16_ref[...] = wp_ref[...].astype(jnp.bfloat16)

    xb = xb_ref[...]
    o0 = jnp.dot(xb, wp16_ref[0], preferred_element_type=jnp.float32)
    o1 = jnp.dot(xb, wp16_ref[1], preferred_element_type=jnp.float32)
    m = (idx_ref[...] > 0)
    out_ref[...] = jnp.where(m, o1 + bp_ref[1:2, :], o0 + bp_ref[0:1, :])


def kernel(x, W1, b1, g1, be1, W2, b2, g2, be2, W3, b3, Wp, bp):
    Bx, Nx, D = x.shape
    T = Bx * Nx
    C, _, P = Wp.shape
    H1 = W1.shape[1]
    xr = x.reshape(T, D)
    TB = 256
    nblk = T // TB

    xb16, idx = pl.pallas_call(
        _classifier_kernel,
        grid=(nblk,),
        in_specs=[
            pl.BlockSpec((TB, D), lambda i: (i, 0)),
            pl.BlockSpec((D, H1), lambda i: (0, 0)),
        ] + [pl.BlockSpec(None, lambda i: (0, 0))] * 9,
        out_specs=[
            pl.BlockSpec((TB, D), lambda i: (i, 0)),
            pl.BlockSpec((T, 1), lambda i: (0, 0)),
        ],
        out_shape=[
            jax.ShapeDtypeStruct((T, D), jnp.bfloat16),
            jax.ShapeDtypeStruct((T, 1), jnp.int32),
        ],
        scratch_shapes=[pltpu.VMEM((T, H1), jnp.float32)],
    )(xr, W1, b1.reshape(1, -1), g1.reshape(1, -1), be1.reshape(1, -1),
      W2, b2.reshape(1, -1), g2.reshape(1, -1), be2.reshape(1, -1),
      W3.reshape(1, -1), b3.reshape(1, -1))

    out = pl.pallas_call(
        _heads_kernel,
        grid=(nblk,),
        in_specs=[
            pl.BlockSpec((TB, D), lambda i: (i, 0)),
            pl.BlockSpec((TB, 1), lambda i: (i, 0)),
            pl.BlockSpec((C, D, P), lambda i: (0, 0, 0)),
            pl.BlockSpec((C, P), lambda i: (0, 0)),
        ],
        out_specs=pl.BlockSpec((TB, P), lambda i: (i, 0)),
        out_shape=jax.ShapeDtypeStruct((T, P), jnp.float32),
        scratch_shapes=[pltpu.VMEM((C, D, P), jnp.bfloat16)],
    )(xb16, idx, Wp, bp)

    return out.reshape(Bx, Nx, P)


# speedup vs baseline: 1.1858x; 1.0711x over previous
import jax
import jax.numpy as jnp
from jax.experimental import pallas as pl
from jax.experimental.pallas import tpu as pltpu


def _fused_kernel(x_ref, wp_ref, w1_ref, b1_ref, g1_ref, be1_ref,
                  w2_ref, b2_ref, g2_ref, be2_ref, w3_ref, b3_ref, bp_ref,
                  out_ref, x16_ref, wp16_ref, h1_ref, idx_ref):
    i = pl.program_id(0)
    nblk = pl.num_programs(0) // 2
    tb = x_ref.shape[0]
    dchunk = wp_ref.shape[1]

    @pl.when(i < nblk)
    def _classify_step():
        x16 = x_ref[...].astype(jnp.bfloat16)
        x16_ref[pl.ds(i * tb, tb), :] = x16
        wp16_ref[:, pl.ds(i * dchunk, dchunk), :] = wp_ref[...].astype(jnp.bfloat16)
        h1 = jnp.dot(x16, w1_ref[...].astype(jnp.bfloat16),
                     preferred_element_type=jnp.float32)
        h1_ref[pl.ds(i * tb, tb), :] = h1

    @pl.when(i == nblk - 1)
    def _finish_classifier():
        h = h1_ref[...] + b1_ref[...]
        mu = jnp.mean(h, axis=0, keepdims=True)
        var = jnp.mean((h - mu) ** 2, axis=0, keepdims=True)
        h = (h - mu) / jnp.sqrt(var + 1e-5) * g1_ref[...] + be1_ref[...]
        h = jnp.maximum(h, 0.0)
        h = jnp.dot(h.astype(jnp.bfloat16), w2_ref[...].astype(jnp.bfloat16),
                    preferred_element_type=jnp.float32)
        h = h + b2_ref[...]
        mu = jnp.mean(h, axis=0, keepdims=True)
        var = jnp.mean((h - mu) ** 2, axis=0, keepdims=True)
        h = (h - mu) / jnp.sqrt(var + 1e-5) * g2_ref[...] + be2_ref[...]
        h = jnp.maximum(h, 0.0)
        h16 = h.astype(jnp.bfloat16).astype(jnp.float32)
        w3 = w3_ref[...].astype(jnp.bfloat16).astype(jnp.float32)
        v = jnp.sum(h16 * w3, axis=1, keepdims=True) + b3_ref[...]
        z = jax.nn.sigmoid(v)
        idx_ref[...] = jnp.clip(jnp.round(z), 0.0, 1.0).astype(jnp.int32)

    @pl.when(i >= nblk)
    def _head_step():
        j = i - nblk
        xb = x16_ref[pl.ds(j * tb, tb), :]
        o0 = jnp.dot(xb, wp16_ref[0], preferred_element_type=jnp.float32)
        o1 = jnp.dot(xb, wp16_ref[1], preferred_element_type=jnp.float32)
        m = (idx_ref[pl.ds(j * tb, tb), :] > 0)
        out_ref[...] = jnp.where(m, o1 + bp_ref[1:2, :], o0 + bp_ref[0:1, :])


def kernel(x, W1, b1, g1, be1, W2, b2, g2, be2, W3, b3, Wp, bp):
    Bx, Nx, D = x.shape
    T = Bx * Nx
    C, _, P = Wp.shape
    H1 = W1.shape[1]
    xr = x.reshape(T, D)
    TB = 256
    nblk = T // TB
    DCH = D // nblk

    out = pl.pallas_call(
        _fused_kernel,
        grid=(2 * nblk,),
        in_specs=[
            pl.BlockSpec((TB, D), lambda i: (jnp.minimum(i, nblk - 1), 0)),
            pl.BlockSpec((C, DCH, P), lambda i: (0, jnp.minimum(i, nblk - 1), 0)),
            pl.BlockSpec((D, H1), lambda i: (0, 0)),
        ] + [pl.BlockSpec(None, lambda i: (0, 0))] * 10,
        out_specs=pl.BlockSpec(
            (TB, P), lambda i: (jnp.maximum(i - nblk, 0), 0)),
        out_shape=jax.ShapeDtypeStruct((T, P), jnp.float32),
        scratch_shapes=[
            pltpu.VMEM((T, D), jnp.bfloat16),
            pltpu.VMEM((C, D, P), jnp.bfloat16),
            pltpu.VMEM((T, H1), jnp.float32),
            pltpu.VMEM((T, 1), jnp.int32),
        ],
    )(xr, Wp, W1, b1.reshape(1, -1), g1.reshape(1, -1), be1.reshape(1, -1),
      W2, b2.reshape(1, -1), g2.reshape(1, -1), be2.reshape(1, -1),
      W3.reshape(1, -1), b3.reshape(1, -1), bp)

    return out.reshape(Bx, Nx, P)


# speedup vs baseline: 1.5411x; 1.2996x over previous
import jax
import jax.numpy as jnp
from jax.experimental import pallas as pl
from jax.experimental.pallas import tpu as pltpu


def _fused_kernel(x_ref, wp_ref, w1_ref, b1_ref, g1_ref, be1_ref,
                  w2_ref, b2_ref, g2_ref, be2_ref, w3_ref, b3_ref, bp_ref,
                  out_ref, x16_ref, wp16_ref, h1_ref, idx_ref):
    i = pl.program_id(0)
    nblk = pl.num_programs(0) // 2
    tb = x_ref.shape[1]
    dchunk = wp_ref.shape[1]

    @pl.when(i < nblk)
    def _classify_step():
        x16 = x_ref[0].astype(jnp.bfloat16)
        x16_ref[pl.ds(i * tb, tb), :] = x16
        wp16_ref[:, pl.ds(i * dchunk, dchunk), :] = wp_ref[...].astype(jnp.bfloat16)
        h1 = jnp.dot(x16, w1_ref[...].astype(jnp.bfloat16),
                     preferred_element_type=jnp.float32)
        h1_ref[pl.ds(i * tb, tb), :] = h1

    @pl.when(i == nblk - 1)
    def _finish_classifier():
        h = h1_ref[...] + b1_ref[...]
        mu = jnp.mean(h, axis=0, keepdims=True)
        var = jnp.mean((h - mu) ** 2, axis=0, keepdims=True)
        h = (h - mu) / jnp.sqrt(var + 1e-5) * g1_ref[...] + be1_ref[...]
        h = jnp.maximum(h, 0.0)
        h = jnp.dot(h.astype(jnp.bfloat16), w2_ref[...].astype(jnp.bfloat16),
                    preferred_element_type=jnp.float32)
        h = h + b2_ref[...]
        mu = jnp.mean(h, axis=0, keepdims=True)
        var = jnp.mean((h - mu) ** 2, axis=0, keepdims=True)
        h = (h - mu) / jnp.sqrt(var + 1e-5) * g2_ref[...] + be2_ref[...]
        h = jnp.maximum(h, 0.0)
        h16 = h.astype(jnp.bfloat16).astype(jnp.float32)
        w3 = w3_ref[...].astype(jnp.bfloat16).astype(jnp.float32)
        v = jnp.sum(h16 * w3, axis=1, keepdims=True) + b3_ref[...]
        z = jax.nn.sigmoid(v)
        idx_ref[...] = jnp.clip(jnp.round(z), 0.0, 1.0).astype(jnp.int32)

    @pl.when(i >= nblk)
    def _head_step():
        j = i - nblk
        xb = x16_ref[pl.ds(j * tb, tb), :]
        o0 = jnp.dot(xb, wp16_ref[0], preferred_element_type=jnp.float32)
        o1 = jnp.dot(xb, wp16_ref[1], preferred_element_type=jnp.float32)
        m = (idx_ref[pl.ds(j * tb, tb), :] > 0)
        out_ref[0] = jnp.where(m, o1 + bp_ref[1:2, :], o0 + bp_ref[0:1, :])


def kernel(x, W1, b1, g1, be1, W2, b2, g2, be2, W3, b3, Wp, bp):
    Bx, Nx, D = x.shape
    T = Bx * Nx
    C, _, P = Wp.shape
    H1 = W1.shape[1]
    TB = 256
    nblk = T // TB
    DCH = D // nblk
    nb = Nx // TB

    def _xmap(i):
        j = jnp.minimum(i, nblk - 1)
        return (j // nb, j % nb, 0)

    def _omap(i):
        j = jnp.maximum(i - nblk, 0)
        return (j // nb, j % nb, 0)

    out = pl.pallas_call(
        _fused_kernel,
        grid=(2 * nblk,),
        in_specs=[
            pl.BlockSpec((1, TB, D), _xmap),
            pl.BlockSpec((C, DCH, P), lambda i: (0, jnp.minimum(i, nblk - 1), 0)),
            pl.BlockSpec((D, H1), lambda i: (0, 0)),
        ] + [pl.BlockSpec(None, lambda i: (0, 0))] * 10,
        out_specs=pl.BlockSpec((1, TB, P), _omap),
        out_shape=jax.ShapeDtypeStruct((Bx, Nx, P), jnp.float32),
        scratch_shapes=[
            pltpu.VMEM((T, D), jnp.bfloat16),
            pltpu.VMEM((C, D, P), jnp.bfloat16),
            pltpu.VMEM((T, H1), jnp.float32),
            pltpu.VMEM((T, 1), jnp.int32),
        ],
    )(x, Wp, W1, b1.reshape(1, -1), g1.reshape(1, -1), be1.reshape(1, -1),
      W2, b2.reshape(1, -1), g2.reshape(1, -1), be2.reshape(1, -1),
      W3.reshape(1, -1), b3.reshape(1, -1), bp)

    return out
